# baseline (speedup 1.0000x reference)
import functools
import jax
import jax.numpy as jnp
from jax import lax
from jax.experimental import pallas as pl
from jax.experimental.pallas import tpu as pltpu
from jax.experimental.pallas import tpu_sc as plsc

_R, _C = 4096, 2048
_NC, _NS = 2, 16
_NW = _NC * _NS
_SC_ROWS = 2048
_RPW = _SC_ROWS // _NW
_CR = 8
_NCHUNK = _RPW // _CR
_TC_BR = 1024


def _sc_body(x_hbm, o_hbm, bufs_in, bufs_out, sems_in, sems_out):
    wid = lax.axis_index("s") * _NC + lax.axis_index("c")
    base_row = wid * _RPW
    lane = lax.iota(jnp.int32, 16)
    idx_swap = lane ^ 1
    even = (lane % 2) == 0

    def row0(k):
        return base_row + k * _CR

    def compute(slot):
        for r in range(_CR):
            @plsc.parallel_loop(0, _C, step=16, unroll=8)
            def _(i):
                v = bufs_in[slot, r, pl.ds(i, 16)]
                p = v[idx_swap]
                lo = jnp.minimum(v, p)
                hi = jnp.maximum(v, p)
                bufs_out[slot, r, pl.ds(i, 16)] = jnp.where(even, lo, hi)

    pltpu.make_async_copy(
        x_hbm.at[pl.ds(row0(0), _CR), :], bufs_in.at[0], sems_in.at[0]
    ).start()

    def body(k, carry):
        slot = lax.rem(k, 2)
        nxt = lax.rem(k + 1, 2)

        @pl.when(k + 1 < _NCHUNK)
        def _():
            pltpu.make_async_copy(
                x_hbm.at[pl.ds(row0(k + 1), _CR), :], bufs_in.at[nxt],
                sems_in.at[nxt],
            ).start()

        pltpu.make_async_copy(
            x_hbm.at[pl.ds(row0(k), _CR), :], bufs_in.at[slot], sems_in.at[slot]
        ).wait()

        @pl.when(k >= 2)
        def _():
            pltpu.make_async_copy(
                bufs_out.at[slot], o_hbm.at[pl.ds(row0(k - 2), _CR), :],
                sems_out.at[slot],
            ).wait()

        compute(slot)

        pltpu.make_async_copy(
            bufs_out.at[slot], o_hbm.at[pl.ds(row0(k), _CR), :], sems_out.at[slot]
        ).start()
        return carry

    lax.fori_loop(0, _NCHUNK, body, 0)

    pltpu.make_async_copy(
        bufs_out.at[_NCHUNK % 2], o_hbm.at[pl.ds(row0(_NCHUNK - 2), _CR), :],
        sems_out.at[_NCHUNK % 2],
    ).wait()
    pltpu.make_async_copy(
        bufs_out.at[(_NCHUNK - 1) % 2],
        o_hbm.at[pl.ds(row0(_NCHUNK - 1), _CR), :],
        sems_out.at[(_NCHUNK - 1) % 2],
    ).wait()


def _twosort_sc(x):
    mesh = plsc.VectorSubcoreMesh(core_axis_name="c", subcore_axis_name="s")
    return pl.kernel(
        _sc_body,
        out_type=jax.ShapeDtypeStruct((_SC_ROWS, _C), jnp.float32),
        mesh=mesh,
        scratch_types=[
            pltpu.VMEM((2, _CR, _C), jnp.float32),
            pltpu.VMEM((2, _CR, _C), jnp.float32),
            pltpu.SemaphoreType.DMA((2,)),
            pltpu.SemaphoreType.DMA((2,)),
        ],
    )(x)


def _tc_block(x_ref, o_ref):
    v = x_ref[...]
    nb = v.shape[1] // 128
    lane = lax.broadcasted_iota(jnp.int32, (v.shape[0], 128), 1)
    even = (lane % 2) == 0
    swap = lane ^ 1
    outs = []
    for g in range(nb):
        s = v[:, g * 128:(g + 1) * 128]
        p = jnp.take_along_axis(s, swap, axis=1)
        outs.append(jnp.where(even, jnp.minimum(s, p), jnp.maximum(s, p)))
    o_ref[...] = jnp.concatenate(outs, axis=1)


def _twosort_tc(x):
    nrows = _R - _SC_ROWS
    return pl.pallas_call(
        _tc_block,
        out_shape=jax.ShapeDtypeStruct((nrows, _C), x.dtype),
        grid=(nrows // _TC_BR,),
        in_specs=[
            pl.BlockSpec((_TC_BR, _C), lambda i: (_SC_ROWS // _TC_BR + i, 0))
        ],
        out_specs=pl.BlockSpec((_TC_BR, _C), lambda i: (i, 0)),
    )(x)


def _twosort_tc_full(x):
    return pl.pallas_call(
        _tc_block,
        out_shape=jax.ShapeDtypeStruct((_R, _C), x.dtype),
        grid=(_R // _TC_BR,),
        in_specs=[pl.BlockSpec((_TC_BR, _C), lambda i: (i, 0))],
        out_specs=pl.BlockSpec((_TC_BR, _C), lambda i: (i, 0)),
    )(x)


_NBUF = 8
_MCR = 256
_MNCH = _R // _MCR


def _tc_manual_body(x_any, o_any, bufs_in, bufs_out, sins, souts):
    lane = lax.broadcasted_iota(jnp.int32, (_MCR, 128), 1)
    even = (lane % 2) == 0
    swap = lane ^ 1

    def in_copy(k, slot):
        return pltpu.make_async_copy(
            x_any.at[pl.ds(k * _MCR, _MCR), :], bufs_in.at[slot], sins.at[slot]
        )

    def out_copy(k, slot):
        return pltpu.make_async_copy(
            bufs_out.at[slot], o_any.at[pl.ds(k * _MCR, _MCR), :], souts.at[slot]
        )

    for b in range(_NBUF - 1):
        in_copy(b, b).start()

    def body(k, carry):
        slot = lax.rem(k, _NBUF)

        @pl.when(k + _NBUF - 1 < _MNCH)
        def _():
            in_copy(k + _NBUF - 1, lax.rem(k + _NBUF - 1, _NBUF)).start()

        in_copy(k, slot).wait()

        @pl.when(k >= _NBUF)
        def _():
            out_copy(k - _NBUF, slot).wait()

        v = bufs_in[slot]
        outs = []
        for g in range(_C // 128):
            s = v[:, g * 128:(g + 1) * 128]
            p = jnp.take_along_axis(s, swap, axis=1)
            outs.append(
                jnp.where(even, jnp.minimum(s, p), jnp.maximum(s, p))
            )
        bufs_out[slot] = jnp.concatenate(outs, axis=1)

        out_copy(k, slot).start()
        return carry

    lax.fori_loop(0, _MNCH, body, 0)

    for b in range(_NBUF):
        k = _MNCH - _NBUF + b
        out_copy(k, k % _NBUF).wait()


def _twosort_tc_manual(x):
    return pl.pallas_call(
        _tc_manual_body,
        out_shape=jax.ShapeDtypeStruct((_R, _C), x.dtype),
        in_specs=[pl.BlockSpec(memory_space=pl.ANY)],
        out_specs=pl.BlockSpec(memory_space=pl.ANY),
        scratch_shapes=[
            pltpu.VMEM((_NBUF, _MCR, _C), jnp.float32),
            pltpu.VMEM((_NBUF, _MCR, _C), jnp.float32),
            pltpu.SemaphoreType.DMA((_NBUF,)),
            pltpu.SemaphoreType.DMA((_NBUF,)),
        ],
    )(x)


def _copy_block(x_ref, o_ref):
    o_ref[...] = x_ref[...]


def _copy_probe(x):
    return pl.pallas_call(
        _copy_block,
        out_shape=jax.ShapeDtypeStruct((_R, _C), x.dtype),
        grid=(_R // _TC_BR,),
        in_specs=[pl.BlockSpec((_TC_BR, _C), lambda i: (i, 0))],
        out_specs=pl.BlockSpec((_TC_BR, _C), lambda i: (i, 0)),
    )(x)


@jax.jit
def _twosort(x):
    return _twosort_tc_manual(x)


def kernel(x):
    return _twosort(x)


# speedup vs baseline: 1.0249x; 1.0249x over previous
import functools
import jax
import jax.numpy as jnp
from jax import lax
from jax.experimental import pallas as pl
from jax.experimental.pallas import tpu as pltpu
from jax.experimental.pallas import tpu_sc as plsc

_R, _C = 4096, 2048
_NC, _NS = 2, 16
_NW = _NC * _NS
_SC_ROWS = 2048
_RPW = _SC_ROWS // _NW
_CR = 8
_NCHUNK = _RPW // _CR
_TC_BR = 1024


def _sc_body(x_hbm, o_hbm, bufs_in, bufs_out, sems_in, sems_out):
    wid = lax.axis_index("s") * _NC + lax.axis_index("c")
    base_row = wid * _RPW
    lane = lax.iota(jnp.int32, 16)
    idx_swap = lane ^ 1
    even = (lane % 2) == 0

    def row0(k):
        return base_row + k * _CR

    def compute(slot):
        for r in range(_CR):
            @plsc.parallel_loop(0, _C, step=16, unroll=8)
            def _(i):
                v = bufs_in[slot, r, pl.ds(i, 16)]
                p = v[idx_swap]
                lo = jnp.minimum(v, p)
                hi = jnp.maximum(v, p)
                bufs_out[slot, r, pl.ds(i, 16)] = jnp.where(even, lo, hi)

    pltpu.make_async_copy(
        x_hbm.at[pl.ds(row0(0), _CR), :], bufs_in.at[0], sems_in.at[0]
    ).start()

    def body(k, carry):
        slot = lax.rem(k, 2)
        nxt = lax.rem(k + 1, 2)

        @pl.when(k + 1 < _NCHUNK)
        def _():
            pltpu.make_async_copy(
                x_hbm.at[pl.ds(row0(k + 1), _CR), :], bufs_in.at[nxt],
                sems_in.at[nxt],
            ).start()

        pltpu.make_async_copy(
            x_hbm.at[pl.ds(row0(k), _CR), :], bufs_in.at[slot], sems_in.at[slot]
        ).wait()

        @pl.when(k >= 2)
        def _():
            pltpu.make_async_copy(
                bufs_out.at[slot], o_hbm.at[pl.ds(row0(k - 2), _CR), :],
                sems_out.at[slot],
            ).wait()

        compute(slot)

        pltpu.make_async_copy(
            bufs_out.at[slot], o_hbm.at[pl.ds(row0(k), _CR), :], sems_out.at[slot]
        ).start()
        return carry

    lax.fori_loop(0, _NCHUNK, body, 0)

    pltpu.make_async_copy(
        bufs_out.at[_NCHUNK % 2], o_hbm.at[pl.ds(row0(_NCHUNK - 2), _CR), :],
        sems_out.at[_NCHUNK % 2],
    ).wait()
    pltpu.make_async_copy(
        bufs_out.at[(_NCHUNK - 1) % 2],
        o_hbm.at[pl.ds(row0(_NCHUNK - 1), _CR), :],
        sems_out.at[(_NCHUNK - 1) % 2],
    ).wait()


def _twosort_sc(x):
    mesh = plsc.VectorSubcoreMesh(core_axis_name="c", subcore_axis_name="s")
    return pl.kernel(
        _sc_body,
        out_type=jax.ShapeDtypeStruct((_SC_ROWS, _C), jnp.float32),
        mesh=mesh,
        scratch_types=[
            pltpu.VMEM((2, _CR, _C), jnp.float32),
            pltpu.VMEM((2, _CR, _C), jnp.float32),
            pltpu.SemaphoreType.DMA((2,)),
            pltpu.SemaphoreType.DMA((2,)),
        ],
    )(x)


def _tc_block(x_ref, o_ref):
    v = x_ref[...]
    nb = v.shape[1] // 128
    lane = lax.broadcasted_iota(jnp.int32, (v.shape[0], 128), 1)
    even = (lane % 2) == 0
    swap = lane ^ 1
    outs = []
    for g in range(nb):
        s = v[:, g * 128:(g + 1) * 128]
        p = jnp.take_along_axis(s, swap, axis=1)
        outs.append(jnp.where(even, jnp.minimum(s, p), jnp.maximum(s, p)))
    o_ref[...] = jnp.concatenate(outs, axis=1)


def _twosort_tc(x):
    nrows = _R - _SC_ROWS
    return pl.pallas_call(
        _tc_block,
        out_shape=jax.ShapeDtypeStruct((nrows, _C), x.dtype),
        grid=(nrows // _TC_BR,),
        in_specs=[
            pl.BlockSpec((_TC_BR, _C), lambda i: (_SC_ROWS // _TC_BR + i, 0))
        ],
        out_specs=pl.BlockSpec((_TC_BR, _C), lambda i: (i, 0)),
    )(x)


def _twosort_tc_full(x):
    return pl.pallas_call(
        _tc_block,
        out_shape=jax.ShapeDtypeStruct((_R, _C), x.dtype),
        grid=(_R // _TC_BR,),
        in_specs=[pl.BlockSpec((_TC_BR, _C), lambda i: (i, 0))],
        out_specs=pl.BlockSpec((_TC_BR, _C), lambda i: (i, 0)),
    )(x)


_NBUF = 3
_MCR = 1024
_MNCH = _R // _MCR


def _tc_manual_body(x_any, o_any, bufs_in, bufs_out, sins, souts):
    lane = lax.broadcasted_iota(jnp.int32, (_MCR, 128), 1)
    even = (lane % 2) == 0
    swap = lane ^ 1

    def in_copy(k, slot):
        return pltpu.make_async_copy(
            x_any.at[pl.ds(k * _MCR, _MCR), :], bufs_in.at[slot], sins.at[slot]
        )

    def out_copy(k, slot):
        return pltpu.make_async_copy(
            bufs_out.at[slot], o_any.at[pl.ds(k * _MCR, _MCR), :], souts.at[slot]
        )

    for b in range(_NBUF - 1):
        in_copy(b, b).start()

    def body(k, carry):
        slot = lax.rem(k, _NBUF)

        @pl.when(k + _NBUF - 1 < _MNCH)
        def _():
            in_copy(k + _NBUF - 1, lax.rem(k + _NBUF - 1, _NBUF)).start()

        in_copy(k, slot).wait()

        @pl.when(k >= _NBUF)
        def _():
            out_copy(k - _NBUF, slot).wait()

        v = bufs_in[slot]
        outs = []
        for g in range(_C // 128):
            s = v[:, g * 128:(g + 1) * 128]
            p = jnp.take_along_axis(s, swap, axis=1)
            outs.append(
                jnp.where(even, jnp.minimum(s, p), jnp.maximum(s, p))
            )
        bufs_out[slot] = jnp.concatenate(outs, axis=1)

        out_copy(k, slot).start()
        return carry

    lax.fori_loop(0, _MNCH, body, 0)

    for b in range(_NBUF):
        k = _MNCH - _NBUF + b
        out_copy(k, k % _NBUF).wait()


def _twosort_tc_manual(x):
    return pl.pallas_call(
        _tc_manual_body,
        out_shape=jax.ShapeDtypeStruct((_R, _C), x.dtype),
        in_specs=[pl.BlockSpec(memory_space=pl.ANY)],
        out_specs=pl.BlockSpec(memory_space=pl.ANY),
        scratch_shapes=[
            pltpu.VMEM((_NBUF, _MCR, _C), jnp.float32),
            pltpu.VMEM((_NBUF, _MCR, _C), jnp.float32),
            pltpu.SemaphoreType.DMA((_NBUF,)),
            pltpu.SemaphoreType.DMA((_NBUF,)),
        ],
    )(x)


def _copy_block(x_ref, o_ref):
    o_ref[...] = x_ref[...]


def _copy_probe(x):
    return pl.pallas_call(
        _copy_block,
        out_shape=jax.ShapeDtypeStruct((_R, _C), x.dtype),
        grid=(_R // _TC_BR,),
        in_specs=[pl.BlockSpec((_TC_BR, _C), lambda i: (i, 0))],
        out_specs=pl.BlockSpec((_TC_BR, _C), lambda i: (i, 0)),
    )(x)


@jax.jit
def _twosort(x):
    return _twosort_tc_manual(x)


def kernel(x):
    return _twosort(x)
